# baseline (speedup 1.0000x reference)
import functools

import jax
import jax.numpy as jnp
from jax import lax
from jax.experimental import pallas as pl
from jax.experimental.pallas import tpu as pltpu
from jax.experimental.pallas import tpu_sc as plsc

ENT = 100000
REL = 1000
D = 128
H = D // 2
B = 16384
LMBDA = 0.001

NC = 2
NS = 16
L = 16
NW = NC * NS
BW = B // NW
C = 32
NCHUNK = BW // C
NBUF = 2


def _sc_body(h_hbm, t_hbm, r_hbm, e1_hbm, e2_hbm, e_hbm, r1_hbm, r2_hbm,
             rel_hbm, res_out, ssh_out, ssd_out,
             hi_v, ti_v, ri_v, hrow_v, trow_v, rrow_v, res_v, ss_v,
             e1h_v, e2h_v, eh_v, e1t_v, e2t_v, et_v, r1_v, r2_v, rel_v,
             sem0, sem1):
    wid = lax.axis_index("s") * NC + lax.axis_index("c")
    base = wid * BW

    pltpu.sync_copy(h_hbm.at[pl.ds(base, BW)], hi_v)
    pltpu.sync_copy(t_hbm.at[pl.ds(base, BW)], ti_v)
    pltpu.sync_copy(r_hbm.at[pl.ds(base, BW)], ri_v)

    for i in range(BW // L):
        s = pl.ds(i * L, L)
        hrow_v[s] = hi_v[s] >> 1
        trow_v[s] = ti_v[s] >> 1
        rrow_v[s] = ri_v[s] >> 1

    sems = (sem0, sem1)

    def issue(g, slot):
        cb = g * C
        hr = hrow_v.at[pl.ds(cb, C)]
        tr = trow_v.at[pl.ds(cb, C)]
        rr = rrow_v.at[pl.ds(cb, C)]
        hi = hi_v.at[pl.ds(cb, C)]
        ti = ti_v.at[pl.ds(cb, C)]
        ri = ri_v.at[pl.ds(cb, C)]
        descs = []
        for tbl, idx, buf in ((e1_hbm, hr, e1h_v), (e2_hbm, hr, e2h_v),
                              (e_hbm, hi, eh_v), (e1_hbm, tr, e1t_v),
                              (e2_hbm, tr, e2t_v), (e_hbm, ti, et_v),
                              (r1_hbm, rr, r1_v), (r2_hbm, rr, r2_v),
                              (rel_hbm, ri, rel_v)):
            descs.append(pltpu.async_copy(tbl.at[idx], buf.at[slot], sems[slot]))
        return descs

    pend = [None] * NBUF
    pend[0] = issue(0, 0)
    ssh = jnp.zeros((L,), jnp.float32)
    ssd = jnp.zeros((L,), jnp.float32)
    lane = lax.iota(jnp.int32, L)
    zero = jnp.zeros((L,), jnp.int32)

    for g in range(NCHUNK):
        slot = g % NBUF
        if g + 1 < NCHUNK:
            pend[(g + 1) % NBUF] = issue(g + 1, (g + 1) % NBUF)
        for dsc in pend[slot]:
            dsc.wait()

        cb = g * C
        for grp in range(C // L):
            c0 = grp * L
            hp = hi_v[pl.ds(cb + c0, L)] & 1
            tp = ti_v[pl.ds(cb + c0, L)] & 1
            rp = ri_v[pl.ds(cb + c0, L)] & 1
            samp = c0 + lane
            hoff = hp * H
            toff = tp * H
            roff = rp * H

            e1h_s = e1h_v.at[slot]
            e2h_s = e2h_v.at[slot]
            eh_s = eh_v.at[slot]
            e1t_s = e1t_v.at[slot]
            e2t_s = e2t_v.at[slot]
            et_s = et_v.at[slot]
            r1_s = r1_v.at[slot]
            r2_s = r2_v.at[slot]
            rel_s = rel_v.at[slot]

            def jbody(j, carry, e1h_s=e1h_s, e2h_s=e2h_s, eh_s=eh_s,
                      e1t_s=e1t_s, e2t_s=e2t_s, et_s=et_s, r1_s=r1_s,
                      r2_s=r2_s, rel_s=rel_s, hoff=hoff, toff=toff,
                      roff=roff, samp=samp):
                acc, ssh, ssd = carry
                a1h = plsc.load_gather(e1h_s, [samp, hoff + j])
                a2h = plsc.load_gather(e2h_s, [samp, hoff + j])
                a1t = plsc.load_gather(e1t_s, [samp, toff + j])
                a2t = plsc.load_gather(e2t_s, [samp, toff + j])
                v1 = plsc.load_gather(r1_s, [samp, roff + j])
                v2 = plsc.load_gather(r2_s, [samp, roff + j])
                acc = acc + (a1h * a1t + a2h * a2t) * v1 \
                          + (a1h * a2t - a2h * a1t) * v2
                ssh = ssh + (a1h * a1h + a2h * a2h) + (a1t * a1t + a2t * a2t) \
                          + (v1 * v1 + v2 * v2)
                j2 = j * 2
                ah0 = plsc.load_gather(eh_s, [samp, zero + j2])
                at0 = plsc.load_gather(et_s, [samp, zero + j2])
                vr0 = plsc.load_gather(rel_s, [samp, zero + j2])
                ah1 = plsc.load_gather(eh_s, [samp, zero + j2 + 1])
                at1 = plsc.load_gather(et_s, [samp, zero + j2 + 1])
                vr1 = plsc.load_gather(rel_s, [samp, zero + j2 + 1])
                acc = acc + ah0 * at0 * vr0 + ah1 * at1 * vr1
                ssd = ssd + (ah0 * ah0 + at0 * at0 + vr0 * vr0) \
                          + (ah1 * ah1 + at1 * at1 + vr1 * vr1)
                return (acc, ssh, ssd)

            acc, ssh, ssd = lax.fori_loop(
                0, H, jbody, (jnp.zeros((L,), jnp.float32), ssh, ssd))
            res_v[pl.ds(cb + c0, L)] = acc

    ss_v[0] = ssh
    ss_v[1] = ssd
    pltpu.sync_copy(res_v, res_out.at[pl.ds(base, BW)])
    pltpu.sync_copy(ss_v.at[0], ssh_out.at[pl.ds(wid * L, L)])
    pltpu.sync_copy(ss_v.at[1], ssd_out.at[pl.ds(wid * L, L)])


_sc_kernel = functools.partial(
    pl.kernel,
    out_type=(
        jax.ShapeDtypeStruct((B,), jnp.float32),
        jax.ShapeDtypeStruct((NW * L,), jnp.float32),
        jax.ShapeDtypeStruct((NW * L,), jnp.float32),
    ),
    mesh=plsc.VectorSubcoreMesh(core_axis_name="c", subcore_axis_name="s"),
    compiler_params=pltpu.CompilerParams(needs_layout_passes=False),
    scratch_types=[
        pltpu.VMEM((BW,), jnp.int32),
        pltpu.VMEM((BW,), jnp.int32),
        pltpu.VMEM((BW,), jnp.int32),
        pltpu.VMEM((BW,), jnp.int32),
        pltpu.VMEM((BW,), jnp.int32),
        pltpu.VMEM((BW,), jnp.int32),
        pltpu.VMEM((BW,), jnp.float32),
        pltpu.VMEM((2, L), jnp.float32),
        pltpu.VMEM((NBUF, C, D), jnp.float32),
        pltpu.VMEM((NBUF, C, D), jnp.float32),
        pltpu.VMEM((NBUF, C, D), jnp.float32),
        pltpu.VMEM((NBUF, C, D), jnp.float32),
        pltpu.VMEM((NBUF, C, D), jnp.float32),
        pltpu.VMEM((NBUF, C, D), jnp.float32),
        pltpu.VMEM((NBUF, C, D), jnp.float32),
        pltpu.VMEM((NBUF, C, D), jnp.float32),
        pltpu.VMEM((NBUF, C, D), jnp.float32),
        pltpu.SemaphoreType.DMA,
        pltpu.SemaphoreType.DMA,
    ],
)(_sc_body)


def _finish_body(res_ref, y_ref, ssh_ref, ssd_ref, out_ref):
    x = -(y_ref[...] * res_ref[...])
    sp = jnp.maximum(x, 0.0) + jnp.log(1.0 + jnp.exp(-jnp.abs(x)))
    reg = jnp.sum(ssh_ref[...]) / (B * H) + jnp.sum(ssd_ref[...]) / (B * D)
    loss = jnp.sum(sp) / B + LMBDA * reg
    out_ref[...] = jnp.full((1, 1), loss, jnp.float32)


def kernel(h, t, r, y, E1, E2, E, R1, R2, R):
    h = h.astype(jnp.int32)
    t = t.astype(jnp.int32)
    r = r.astype(jnp.int32)
    e1p = E1.reshape(ENT // 2, D)
    e2p = E2.reshape(ENT // 2, D)
    r1p = R1.reshape(REL // 2, D)
    r2p = R2.reshape(REL // 2, D)
    res, ssh, ssd = _sc_kernel(h, t, r, e1p, e2p, E, r1p, r2p, R)
    loss = pl.pallas_call(
        _finish_body,
        out_shape=jax.ShapeDtypeStruct((1, 1), jnp.float32),
    )(res.reshape(128, 128), y.reshape(128, 128),
      ssh.reshape(4, 128), ssd.reshape(4, 128))
    return loss[0, 0]


# speedup vs baseline: 1.9795x; 1.9795x over previous
import functools

import jax
import jax.numpy as jnp
from jax import lax
from jax.experimental import pallas as pl
from jax.experimental.pallas import tpu as pltpu
from jax.experimental.pallas import tpu_sc as plsc

ENT = 100000
REL = 1000
D = 128
H = D // 2
B = 16384
LMBDA = 0.001

NC = 2
NS = 16
L = 16
NW = NC * NS
BW = B // NW
C = 32
NCHUNK = BW // C
NBUF = 2


def _sc_body(h_hbm, t_hbm, r_hbm, e1_hbm, e2_hbm, e_hbm, r1_hbm, r2_hbm,
             rel_hbm, res_out, ssh_out, ssd_out,
             hi_v, ti_v, ri_v, hrow_v, trow_v, rrow_v, res_v, ss_v, stage_v,
             e1h_v, e2h_v, eh_v, e1t_v, e2t_v, et_v, r1_v, r2_v, rel_v,
             sem0, sem1):
    wid = lax.axis_index("s") * NC + lax.axis_index("c")
    base = wid * BW

    pltpu.sync_copy(h_hbm.at[pl.ds(base, BW)], hi_v.at[pl.ds(0, BW)])
    pltpu.sync_copy(t_hbm.at[pl.ds(base, BW)], ti_v.at[pl.ds(0, BW)])
    pltpu.sync_copy(r_hbm.at[pl.ds(base, BW)], ri_v.at[pl.ds(0, BW)])

    for i in range(BW // L):
        s = pl.ds(i * L, L)
        hrow_v[s] = hi_v[s] >> 1
        trow_v[s] = ti_v[s] >> 1
        rrow_v[s] = ri_v[s] >> 1

    sems = (sem0, sem1)

    def issue(g, slot):
        cb = g * C
        hr = hrow_v.at[pl.ds(cb, C)]
        tr = trow_v.at[pl.ds(cb, C)]
        rr = rrow_v.at[pl.ds(cb, C)]
        hi = hi_v.at[pl.ds(cb, C)]
        ti = ti_v.at[pl.ds(cb, C)]
        ri = ri_v.at[pl.ds(cb, C)]
        descs = []
        for tbl, idx, buf in ((e1_hbm, hr, e1h_v), (e2_hbm, hr, e2h_v),
                              (e_hbm, hi, eh_v), (e1_hbm, tr, e1t_v),
                              (e2_hbm, tr, e2t_v), (e_hbm, ti, et_v),
                              (r1_hbm, rr, r1_v), (r2_hbm, rr, r2_v),
                              (rel_hbm, ri, rel_v)):
            descs.append(pltpu.async_copy(tbl.at[idx], buf.at[slot], sems[slot]))
        return descs

    pend = [None] * NBUF
    pend[0] = issue(0, 0)
    ssh = jnp.zeros((L,), jnp.float32)
    ssd = jnp.zeros((L,), jnp.float32)
    lane = lax.iota(jnp.int32, L)

    for g in range(NCHUNK):
        slot = g % NBUF
        if g + 1 < NCHUNK:
            pend[(g + 1) % NBUF] = issue(g + 1, (g + 1) % NBUF)
        for dsc in pend[slot]:
            dsc.wait()

        cb = g * C

        def body(c, carry, slot=slot, cb=cb):
            ssh, ssd = carry
            hp = (hi_v[pl.ds(cb + c, L)][0] & 1) * H
            tp = (ti_v[pl.ds(cb + c, L)][0] & 1) * H
            rp = (ri_v[pl.ds(cb + c, L)][0] & 1) * H
            acc = jnp.zeros((L,), jnp.float32)
            for k in range(H // L):
                a1h = e1h_v[slot, c, pl.ds(hp + k * L, L)]
                a2h = e2h_v[slot, c, pl.ds(hp + k * L, L)]
                a1t = e1t_v[slot, c, pl.ds(tp + k * L, L)]
                a2t = e2t_v[slot, c, pl.ds(tp + k * L, L)]
                v1 = r1_v[slot, c, pl.ds(rp + k * L, L)]
                v2 = r2_v[slot, c, pl.ds(rp + k * L, L)]
                acc = acc + (a1h * a1t + a2h * a2t) * v1 \
                          + (a1h * a2t - a2h * a1t) * v2
                ssh = ssh + (a1h * a1h + a2h * a2h) + (a1t * a1t + a2t * a2t) \
                          + (v1 * v1 + v2 * v2)
            for k in range(D // L):
                s = pl.ds(k * L, L)
                ah = eh_v[slot, c, s]
                at = et_v[slot, c, s]
                vr = rel_v[slot, c, s]
                acc = acc + ah * at * vr
                ssd = ssd + ah * ah + at * at + vr * vr
            plsc.store_scatter(stage_v, [lane * C + c], acc)
            return (ssh, ssd)

        ssh, ssd = lax.fori_loop(0, C, body, (ssh, ssd))

        for k in range(C // L):
            rv = stage_v[pl.ds(k * L, L)]
            for l in range(1, L):
                rv = rv + stage_v[pl.ds(l * C + k * L, L)]
            res_v[pl.ds(cb + k * L, L)] = rv

    ss_v[0] = ssh
    ss_v[1] = ssd
    pltpu.sync_copy(res_v, res_out.at[pl.ds(base, BW)])
    pltpu.sync_copy(ss_v.at[0], ssh_out.at[pl.ds(wid * L, L)])
    pltpu.sync_copy(ss_v.at[1], ssd_out.at[pl.ds(wid * L, L)])


_sc_kernel = functools.partial(
    pl.kernel,
    out_type=(
        jax.ShapeDtypeStruct((B,), jnp.float32),
        jax.ShapeDtypeStruct((NW * L,), jnp.float32),
        jax.ShapeDtypeStruct((NW * L,), jnp.float32),
    ),
    mesh=plsc.VectorSubcoreMesh(core_axis_name="c", subcore_axis_name="s"),
    compiler_params=pltpu.CompilerParams(needs_layout_passes=False),
    scratch_types=[
        pltpu.VMEM((BW + L,), jnp.int32),
        pltpu.VMEM((BW + L,), jnp.int32),
        pltpu.VMEM((BW + L,), jnp.int32),
        pltpu.VMEM((BW,), jnp.int32),
        pltpu.VMEM((BW,), jnp.int32),
        pltpu.VMEM((BW,), jnp.int32),
        pltpu.VMEM((BW,), jnp.float32),
        pltpu.VMEM((2, L), jnp.float32),
        pltpu.VMEM((L * C,), jnp.float32),
        pltpu.VMEM((NBUF, C, D), jnp.float32),
        pltpu.VMEM((NBUF, C, D), jnp.float32),
        pltpu.VMEM((NBUF, C, D), jnp.float32),
        pltpu.VMEM((NBUF, C, D), jnp.float32),
        pltpu.VMEM((NBUF, C, D), jnp.float32),
        pltpu.VMEM((NBUF, C, D), jnp.float32),
        pltpu.VMEM((NBUF, C, D), jnp.float32),
        pltpu.VMEM((NBUF, C, D), jnp.float32),
        pltpu.VMEM((NBUF, C, D), jnp.float32),
        pltpu.SemaphoreType.DMA,
        pltpu.SemaphoreType.DMA,
    ],
)(_sc_body)


def _finish_body(res_ref, y_ref, ssh_ref, ssd_ref, out_ref):
    x = -(y_ref[...] * res_ref[...])
    sp = jnp.maximum(x, 0.0) + jnp.log(1.0 + jnp.exp(-jnp.abs(x)))
    reg = jnp.sum(ssh_ref[...]) / (B * H) + jnp.sum(ssd_ref[...]) / (B * D)
    loss = jnp.sum(sp) / B + LMBDA * reg
    out_ref[...] = jnp.full((1, 1), loss, jnp.float32)


def kernel(h, t, r, y, E1, E2, E, R1, R2, R):
    h = h.astype(jnp.int32)
    t = t.astype(jnp.int32)
    r = r.astype(jnp.int32)
    e1p = E1.reshape(ENT // 2, D)
    e2p = E2.reshape(ENT // 2, D)
    r1p = R1.reshape(REL // 2, D)
    r2p = R2.reshape(REL // 2, D)
    res, ssh, ssd = _sc_kernel(h, t, r, e1p, e2p, E, r1p, r2p, R)
    loss = pl.pallas_call(
        _finish_body,
        out_shape=jax.ShapeDtypeStruct((1, 1), jnp.float32),
    )(res.reshape(128, 128), y.reshape(128, 128),
      ssh.reshape(4, 128), ssd.reshape(4, 128))
    return loss[0, 0]
